# baseline (speedup 1.0000x reference)
import functools

import jax
import jax.numpy as jnp
from jax import lax
from jax.experimental import pallas as pl
from jax.experimental.pallas import tpu as pltpu
from jax.experimental.pallas import tpu_sc as plsc

NC, NS = 2, 16
NW = NC * NS
K = 128
SW = 128


def _round_up(a, b):
    return -(-a // b) * b


def _pre_body(x_ref, w_ref, b_ref, o_ref):
    o_ref[...] = (
        jnp.dot(x_ref[...], w_ref[...], preferred_element_type=jnp.float32)
        + b_ref[...]
    )


def _pre(xp, WT, b, Bn=1024):
    Np, D = xp.shape
    return pl.pallas_call(
        _pre_body,
        grid=(Np // Bn,),
        in_specs=[
            pl.BlockSpec((Bn, D), lambda i: (i, 0)),
            pl.BlockSpec((D, D), lambda i: (0, 0)),
            pl.BlockSpec((1, D), lambda i: (0, 0)),
        ],
        out_specs=pl.BlockSpec((Bn, D), lambda i: (i, 0)),
        out_shape=jax.ShapeDtypeStruct((Np, D), jnp.float32),
    )(xp, WT, b)


def _gather(x1, row_p, col_p):
    Np, D = x1.shape
    Ep = row_p.shape[0]
    per_w = Ep // NW
    nchunk = per_w // K
    mesh = plsc.VectorSubcoreMesh(core_axis_name="c", subcore_axis_name="s")

    @functools.partial(
        pl.kernel,
        mesh=mesh,
        out_type=[
            jax.ShapeDtypeStruct((Ep, D), jnp.float32),
            jax.ShapeDtypeStruct((Ep, D), jnp.float32),
        ],
        scratch_types=[
            pltpu.VMEM((K,), jnp.int32),
            pltpu.VMEM((K,), jnp.int32),
            pltpu.VMEM((K, D), jnp.float32),
            pltpu.VMEM((K, D), jnp.float32),
            pltpu.SemaphoreType.DMA,
            pltpu.SemaphoreType.DMA,
        ],
    )
    def k(x1_hbm, row_hbm, col_hbm, gr_hbm, gc_hbm, idx_r, idx_c, bufr, bufc,
          semr, semc):
        w = lax.axis_index("s") * NC + lax.axis_index("c")
        base0 = w * per_w

        def body(j, carry):
            base = base0 + j * K
            pltpu.sync_copy(row_hbm.at[pl.ds(base, K)], idx_r)
            pltpu.sync_copy(col_hbm.at[pl.ds(base, K)], idx_c)
            cr = pltpu.async_copy(x1_hbm.at[idx_r], bufr, semr)
            cc = pltpu.async_copy(x1_hbm.at[idx_c], bufc, semc)
            cr.wait()
            cc.wait()
            pltpu.sync_copy(bufr, gr_hbm.at[pl.ds(base, K)])
            pltpu.sync_copy(bufc, gc_hbm.at[pl.ds(base, K)])
            return carry

        lax.fori_loop(0, nchunk, body, 0)

    return k(x1, row_p, col_p)


def _edge_body(gr_ref, gc_ref, e4_ref, ar_ref, ac_ref, ux_ref, ae_ref, ue_ref,
               wa2_ref, we2_ref, ba1_ref, be1_ref, ba2_ref, be2_ref, s_ref,
               ea_ref):
    gr = gr_ref[...]
    gc = gc_ref[...]
    d = gr - gc
    g = jnp.sqrt(jnp.sum(d * d, axis=1, keepdims=True) + 1e-12)
    e4 = e4_ref[...]
    emask = e4[:, 3:4]
    ea = jnp.concatenate([e4[:, :3], g], axis=1)
    h = (
        jnp.dot(gr, ar_ref[...], preferred_element_type=jnp.float32)
        + jnp.dot(gc, ac_ref[...], preferred_element_type=jnp.float32)
        + jnp.dot(ea, ae_ref[...], preferred_element_type=jnp.float32)
        + ba1_ref[...]
    )
    hs = h * jax.nn.sigmoid(h)
    att = jax.nn.sigmoid(
        jnp.dot(hs, wa2_ref[...], preferred_element_type=jnp.float32)
        + ba2_ref[...]
    ) * emask
    hm = (
        jnp.dot(gc - gr, ux_ref[...], preferred_element_type=jnp.float32)
        + jnp.dot(ea, ue_ref[...], preferred_element_type=jnp.float32)
        + be1_ref[...]
    )
    hm = hm * jax.nn.sigmoid(hm)
    msg = jnp.dot(hm, we2_ref[...], preferred_element_type=jnp.float32) + be2_ref[...]
    s_ref[...] = msg * att
    ea_ref[...] = ea


def _edge(Gr, Gc, e4, ArT, AcT, UxT, AeT, UeT, wa2c, We2T, ba1, be1, ba2, be2,
          B=1024):
    Ep, D = Gr.shape
    full = lambda shape: pl.BlockSpec(shape, lambda i: (0, 0))
    return pl.pallas_call(
        _edge_body,
        grid=(Ep // B,),
        in_specs=[
            pl.BlockSpec((B, D), lambda i: (i, 0)),
            pl.BlockSpec((B, D), lambda i: (i, 0)),
            pl.BlockSpec((B, 4), lambda i: (i, 0)),
            full((D, D)), full((D, D)), full((D, D)),
            full((4, D)), full((4, D)), full((D, 1)), full((D, D)),
            full((1, D)), full((1, D)), full((1, 1)), full((1, D)),
        ],
        out_specs=[
            pl.BlockSpec((B, SW), lambda i: (i, 0)),
            pl.BlockSpec((B, 4), lambda i: (i, 0)),
        ],
        out_shape=[
            jax.ShapeDtypeStruct((Ep, SW), jnp.float32),
            jax.ShapeDtypeStruct((Ep, 4), jnp.float32),
        ],
    )(Gr, Gc, e4, ArT, AcT, UxT, AeT, UeT, wa2c, We2T, ba1, be1, ba2, be2)


def _scatter(s_full, row_p, zrows, Npad):
    Ep = s_full.shape[0]
    per_w = Ep // NW
    nchunk = per_w // K
    rows_per = Npad // NS
    mesh = plsc.VectorSubcoreMesh(core_axis_name="c", subcore_axis_name="s")

    @functools.partial(
        pl.kernel,
        mesh=mesh,
        out_type=jax.ShapeDtypeStruct((NC, Npad, SW), jnp.float32),
        scratch_types=[
            pltpu.VMEM((K,), jnp.int32),
            pltpu.VMEM((K, SW), jnp.float32),
            pltpu.VMEM_SHARED((Npad, SW), jnp.float32),
        ],
    )
    def k(s_hbm, row_hbm, z_hbm, out_hbm, idxA, bufA, shared):
        c = lax.axis_index("c")
        s = lax.axis_index("s")

        pltpu.sync_copy(z_hbm, bufA)

        def zbody(j, carry):
            pltpu.sync_copy(bufA, shared.at[pl.ds(s * rows_per + j * K, K)])
            return carry

        lax.fori_loop(0, rows_per // K, zbody, 0)
        plsc.subcore_barrier()

        base0 = (c * NS + s) * per_w

        def body(j, carry):
            base = base0 + j * K
            pltpu.sync_copy(row_hbm.at[pl.ds(base, K)], idxA)
            pltpu.sync_copy(s_hbm.at[pl.ds(base, K)], bufA)
            pltpu.sync_copy(bufA, shared.at[idxA], add=True)
            return carry

        lax.fori_loop(0, nchunk, body, 0)
        plsc.subcore_barrier()

        def obody(j, carry):
            r0 = s * rows_per + j * K
            pltpu.sync_copy(shared.at[pl.ds(r0, K)], bufA)
            pltpu.sync_copy(bufA, out_hbm.at[c, pl.ds(r0, K)])
            return carry

        lax.fori_loop(0, rows_per // K, obody, 0)

    return k(s_full, row_p, zrows)


def _post_body(sp_ref, x1_ref, wn1_ref, bn1_ref, wn2_ref,
               bn2_ref, lns_ref, lnb_ref, o_ref):
    agg = sp_ref[0, :, :] + sp_ref[1, :, :]
    h = jnp.dot(agg, wn1_ref[...], preferred_element_type=jnp.float32) + bn1_ref[...]
    h = h * jax.nn.sigmoid(h)
    out = jnp.dot(h, wn2_ref[...], preferred_element_type=jnp.float32) + bn2_ref[...]
    y = x1_ref[...] + out
    mu = jnp.mean(y, axis=1, keepdims=True)
    yc = y - mu
    var = jnp.mean(yc * yc, axis=1, keepdims=True)
    yn = yc / jnp.sqrt(var + 1e-5) * lns_ref[...] + lnb_ref[...]
    o_ref[...] = yn * jax.nn.sigmoid(yn)


def _post(Sp, x1, Wn1T, bn1, Wn2T, bn2, lns, lnb, Bn=1024):
    Np, D = x1.shape
    full = lambda shape: pl.BlockSpec(shape, lambda i: (0, 0))
    return pl.pallas_call(
        _post_body,
        grid=(Np // Bn,),
        in_specs=[
            pl.BlockSpec((NC, Bn, SW), lambda i: (0, i, 0)),
            pl.BlockSpec((Bn, D), lambda i: (i, 0)),
            full((D, D)), full((1, D)),
            full((D, D)), full((1, D)),
            full((1, D)), full((1, D)),
        ],
        out_specs=pl.BlockSpec((Bn, D), lambda i: (i, 0)),
        out_shape=jax.ShapeDtypeStruct((Np, D), jnp.float32),
    )(Sp, x1, Wn1T, bn1, Wn2T, bn2, lns, lnb)


def kernel(x, edge_attr, edges, node_mask, edge_mask, W_lin, bias, We1, be1,
           We2, be2, Wn1, bn1, Wn2, bn2, Wa1, ba1, Wa2, ba2, ln_scale,
           ln_bias):
    N, D = x.shape
    E = edges.shape[1]
    Np = _round_up(N, NS * K)
    Ep = _round_up(E, NW * K * 2)

    xp = jnp.pad(x, ((0, Np - N), (0, 0)))
    row_p = jnp.pad(edges[0], (0, Ep - E))
    col_p = jnp.pad(edges[1], (0, Ep - E))
    e4 = jnp.pad(
        jnp.concatenate([edge_attr, edge_mask], axis=1), ((0, Ep - E), (0, 0))
    )
    zrows = jnp.zeros((K, SW), jnp.float32)

    ArT = Wa1[:, :D].T
    AcT = Wa1[:, D:2 * D].T
    AeT = Wa1[:, 2 * D:].T
    UxT = We1[:, :D].T
    UeT = We1[:, D:].T
    wa2c = Wa2.T
    ba2r = ba2.reshape(1, 1)
    ba1r = ba1.reshape(1, D)
    be1r = be1.reshape(1, D)
    lns = ln_scale.reshape(1, D)
    lnb = ln_bias.reshape(1, D)

    x1 = _pre(xp, W_lin.T, bias)
    Gr, Gc = _gather(x1, row_p, col_p)
    s_full, ea_p = _edge(Gr, Gc, e4, ArT, AcT, UxT, AeT, UeT, wa2c, We2.T,
                         ba1r, be1r, ba2r, be2.reshape(1, D))
    Sp = _scatter(s_full, row_p, zrows, Np)
    x_out = _post(Sp, x1, Wn1.T, bn1.reshape(1, D),
                  Wn2.T, bn2.reshape(1, D), lns, lnb)

    return (x_out[:N], ea_p[:E], edges, node_mask, edge_mask)


# speedup vs baseline: 1.3374x; 1.3374x over previous
import functools

import jax
import jax.numpy as jnp
from jax import lax
from jax.experimental import pallas as pl
from jax.experimental.pallas import tpu as pltpu
from jax.experimental.pallas import tpu_sc as plsc

NC, NS = 2, 16
NW = NC * NS
K = 128
SW = 128


def _round_up(a, b):
    return -(-a // b) * b


def _pre_body(x_ref, w_ref, b_ref, o_ref):
    o_ref[...] = (
        jnp.dot(x_ref[...], w_ref[...], preferred_element_type=jnp.float32)
        + b_ref[...]
    )


def _pre(xp, WT, b, Bn=1024):
    Np, D = xp.shape
    return pl.pallas_call(
        _pre_body,
        grid=(Np // Bn,),
        in_specs=[
            pl.BlockSpec((Bn, D), lambda i: (i, 0)),
            pl.BlockSpec((D, D), lambda i: (0, 0)),
            pl.BlockSpec((1, D), lambda i: (0, 0)),
        ],
        out_specs=pl.BlockSpec((Bn, D), lambda i: (i, 0)),
        out_shape=jax.ShapeDtypeStruct((Np, D), jnp.float32),
    )(xp, WT, b)


def _gather(x1, row_p, col_p):
    Np, D = x1.shape
    Ep = row_p.shape[0]
    per_w = Ep // NW
    nchunk = per_w // K
    mesh = plsc.VectorSubcoreMesh(core_axis_name="c", subcore_axis_name="s")

    @functools.partial(
        pl.kernel,
        mesh=mesh,
        out_type=[
            jax.ShapeDtypeStruct((Ep, D), jnp.float32),
            jax.ShapeDtypeStruct((Ep, D), jnp.float32),
        ],
        scratch_types=[
            pltpu.VMEM((K,), jnp.int32),
            pltpu.VMEM((K,), jnp.int32),
            pltpu.VMEM((K, D), jnp.float32),
            pltpu.VMEM((K, D), jnp.float32),
            pltpu.SemaphoreType.DMA,
            pltpu.SemaphoreType.DMA,
        ],
    )
    def k(x1_hbm, row_hbm, col_hbm, gr_hbm, gc_hbm, idx_r, idx_c, bufr, bufc,
          semr, semc):
        w = lax.axis_index("s") * NC + lax.axis_index("c")
        base0 = w * per_w

        def body(j, carry):
            base = base0 + j * K
            pltpu.sync_copy(row_hbm.at[pl.ds(base, K)], idx_r)
            pltpu.sync_copy(col_hbm.at[pl.ds(base, K)], idx_c)
            cr = pltpu.async_copy(x1_hbm.at[idx_r], bufr, semr)
            cc = pltpu.async_copy(x1_hbm.at[idx_c], bufc, semc)
            cr.wait()
            cc.wait()
            pltpu.sync_copy(bufr, gr_hbm.at[pl.ds(base, K)])
            pltpu.sync_copy(bufc, gc_hbm.at[pl.ds(base, K)])
            return carry

        lax.fori_loop(0, nchunk, body, 0)

    return k(x1, row_p, col_p)


def _edge_body(gr_ref, gc_ref, e4_ref, ar_ref, ac_ref, ux_ref, ae_ref, ue_ref,
               wa2_ref, we2_ref, ba1_ref, be1_ref, ba2_ref, be2_ref, s_ref,
               ea_ref):
    bf = jnp.bfloat16
    gr = gr_ref[...]
    gc = gc_ref[...]
    d = gr - gc
    g = jnp.sqrt(jnp.sum(d * d, axis=1, keepdims=True) + 1e-12)
    e4 = e4_ref[...]
    emask = e4[:, 3:4]
    ea = jnp.concatenate([e4[:, :3], g], axis=1)
    h = (
        jnp.dot(gr.astype(bf), ar_ref[...], preferred_element_type=jnp.float32)
        + jnp.dot(gc.astype(bf), ac_ref[...], preferred_element_type=jnp.float32)
        + jnp.dot(ea, ae_ref[...], preferred_element_type=jnp.float32)
        + ba1_ref[...]
    )
    hs = h * jax.nn.sigmoid(h)
    att = jax.nn.sigmoid(
        jnp.dot(hs, wa2_ref[...], preferred_element_type=jnp.float32)
        + ba2_ref[...]
    ) * emask
    hm = (
        jnp.dot((gc - gr).astype(bf), ux_ref[...],
                preferred_element_type=jnp.float32)
        + jnp.dot(ea, ue_ref[...], preferred_element_type=jnp.float32)
        + be1_ref[...]
    )
    hm = hm * jax.nn.sigmoid(hm)
    msg = jnp.dot(hm.astype(bf), we2_ref[...],
                  preferred_element_type=jnp.float32) + be2_ref[...]
    s_ref[...] = msg * att
    ea_ref[...] = ea


def _edge(Gr, Gc, e4, ArT, AcT, UxT, AeT, UeT, wa2c, We2T, ba1, be1, ba2, be2,
          B=1024):
    Ep, D = Gr.shape
    full = lambda shape: pl.BlockSpec(shape, lambda i: (0, 0))
    return pl.pallas_call(
        _edge_body,
        grid=(Ep // B,),
        in_specs=[
            pl.BlockSpec((B, D), lambda i: (i, 0)),
            pl.BlockSpec((B, D), lambda i: (i, 0)),
            pl.BlockSpec((B, 4), lambda i: (i, 0)),
            full((D, D)), full((D, D)), full((D, D)),
            full((4, D)), full((4, D)), full((D, 1)), full((D, D)),
            full((1, D)), full((1, D)), full((1, 1)), full((1, D)),
        ],
        out_specs=[
            pl.BlockSpec((B, SW), lambda i: (i, 0)),
            pl.BlockSpec((B, 4), lambda i: (i, 0)),
        ],
        out_shape=[
            jax.ShapeDtypeStruct((Ep, SW), jnp.float32),
            jax.ShapeDtypeStruct((Ep, 4), jnp.float32),
        ],
    )(Gr, Gc, e4, ArT, AcT, UxT, AeT, UeT, wa2c, We2T, ba1, be1, ba2, be2)


def _scatter(s_full, row_p, zrows, Npad):
    Ep = s_full.shape[0]
    per_w = Ep // NW
    nchunk = per_w // K
    rows_per = Npad // NS
    mesh = plsc.VectorSubcoreMesh(core_axis_name="c", subcore_axis_name="s")

    @functools.partial(
        pl.kernel,
        mesh=mesh,
        out_type=jax.ShapeDtypeStruct((NC, Npad, SW), jnp.float32),
        scratch_types=[
            pltpu.VMEM((K,), jnp.int32),
            pltpu.VMEM((K, SW), jnp.float32),
            pltpu.VMEM_SHARED((Npad, SW), jnp.float32),
        ],
    )
    def k(s_hbm, row_hbm, z_hbm, out_hbm, idxA, bufA, shared):
        c = lax.axis_index("c")
        s = lax.axis_index("s")

        pltpu.sync_copy(z_hbm, bufA)

        def zbody(j, carry):
            pltpu.sync_copy(bufA, shared.at[pl.ds(s * rows_per + j * K, K)])
            return carry

        lax.fori_loop(0, rows_per // K, zbody, 0)
        plsc.subcore_barrier()

        base0 = (c * NS + s) * per_w

        def body(j, carry):
            base = base0 + j * K
            pltpu.sync_copy(row_hbm.at[pl.ds(base, K)], idxA)
            pltpu.sync_copy(s_hbm.at[pl.ds(base, K)], bufA)
            pltpu.sync_copy(bufA, shared.at[idxA], add=True)
            return carry

        lax.fori_loop(0, nchunk, body, 0)
        plsc.subcore_barrier()

        def obody(j, carry):
            r0 = s * rows_per + j * K
            pltpu.sync_copy(shared.at[pl.ds(r0, K)], bufA)
            pltpu.sync_copy(bufA, out_hbm.at[c, pl.ds(r0, K)])
            return carry

        lax.fori_loop(0, rows_per // K, obody, 0)

    return k(s_full, row_p, zrows)


def _post_body(sp_ref, x1_ref, wn1_ref, bn1_ref, wn2_ref,
               bn2_ref, lns_ref, lnb_ref, o_ref):
    agg = sp_ref[0, :, :] + sp_ref[1, :, :]
    h = jnp.dot(agg, wn1_ref[...], preferred_element_type=jnp.float32) + bn1_ref[...]
    h = h * jax.nn.sigmoid(h)
    out = jnp.dot(h, wn2_ref[...], preferred_element_type=jnp.float32) + bn2_ref[...]
    y = x1_ref[...] + out
    mu = jnp.mean(y, axis=1, keepdims=True)
    yc = y - mu
    var = jnp.mean(yc * yc, axis=1, keepdims=True)
    yn = yc / jnp.sqrt(var + 1e-5) * lns_ref[...] + lnb_ref[...]
    o_ref[...] = yn * jax.nn.sigmoid(yn)


def _post(Sp, x1, Wn1T, bn1, Wn2T, bn2, lns, lnb, Bn=1024):
    Np, D = x1.shape
    full = lambda shape: pl.BlockSpec(shape, lambda i: (0, 0))
    return pl.pallas_call(
        _post_body,
        grid=(Np // Bn,),
        in_specs=[
            pl.BlockSpec((NC, Bn, SW), lambda i: (0, i, 0)),
            pl.BlockSpec((Bn, D), lambda i: (i, 0)),
            full((D, D)), full((1, D)),
            full((D, D)), full((1, D)),
            full((1, D)), full((1, D)),
        ],
        out_specs=pl.BlockSpec((Bn, D), lambda i: (i, 0)),
        out_shape=jax.ShapeDtypeStruct((Np, D), jnp.float32),
    )(Sp, x1, Wn1T, bn1, Wn2T, bn2, lns, lnb)


def kernel(x, edge_attr, edges, node_mask, edge_mask, W_lin, bias, We1, be1,
           We2, be2, Wn1, bn1, Wn2, bn2, Wa1, ba1, Wa2, ba2, ln_scale,
           ln_bias):
    N, D = x.shape
    E = edges.shape[1]
    Np = _round_up(N, NS * K)
    Ep = _round_up(E, NW * K)

    xp = jnp.pad(x, ((0, Np - N), (0, 0)))
    row_p = jnp.pad(edges[0], (0, Ep - E))
    col_p = jnp.pad(edges[1], (0, Ep - E))
    e4 = jnp.pad(
        jnp.concatenate([edge_attr, edge_mask], axis=1), ((0, Ep - E), (0, 0))
    )
    zrows = jnp.zeros((K, SW), jnp.float32)

    ArT = Wa1[:, :D].T
    AcT = Wa1[:, D:2 * D].T
    AeT = Wa1[:, 2 * D:].T
    UxT = We1[:, :D].T
    UeT = We1[:, D:].T
    wa2c = Wa2.T
    ba2r = ba2.reshape(1, 1)
    ba1r = ba1.reshape(1, D)
    be1r = be1.reshape(1, D)
    lns = ln_scale.reshape(1, D)
    lnb = ln_bias.reshape(1, D)

    x1 = _pre(xp, W_lin.T, bias)
    Gr, Gc = _gather(x1, row_p, col_p)
    bf = jnp.bfloat16
    s_full, ea_p = _edge(Gr, Gc, e4, ArT.astype(bf), AcT.astype(bf),
                         UxT.astype(bf), AeT, UeT, wa2c, We2.T.astype(bf),
                         ba1r, be1r, ba2r, be2.reshape(1, D))
    Sp = _scatter(s_full, row_p, zrows, Np)
    x_out = _post(Sp, x1, Wn1.T, bn1.reshape(1, D),
                  Wn2.T, bn2.reshape(1, D), lns, lnb)

    return (x_out[:N], ea_p[:E], edges, node_mask, edge_mask)
